# Initial kernel scaffold; baseline (speedup 1.0000x reference)
import jax
import jax.numpy as jnp
from jax.experimental import pallas as pl


def kernel(feats, points, W, b_fc, gamma, beta):
    raise NotImplementedError("write your pallas kernel here")


# speedup vs baseline: 51.9475x; 51.9475x over previous
import functools

import jax
import jax.numpy as jnp
from jax import lax
from jax.experimental import pallas as pl
from jax.experimental.pallas import tpu as pltpu
from jax.experimental.pallas import tpu_sc as plsc

_K = 16


def _fps_body(pts_ref, samp_ref, p2_ref, dist_ref):
    b, d, r, l = pts_ref.shape
    n = r * l
    n_points = samp_ref.shape[1]
    i32 = jnp.int32
    flat3 = (lax.broadcasted_iota(i32, (1, r, l), 1) * l
             + lax.broadcasted_iota(i32, (1, r, l), 2))
    flat4 = flat3.reshape(1, 1, r, l)
    iota_np = lax.broadcasted_iota(i32, (b, n_points), 1)
    iota_np3 = lax.broadcasted_iota(i32, (1, 1, n_points), 2)
    dist_ref[...] = jnp.full((b, r, l), 1e10, jnp.float32)
    pts = pts_ref[...]

    def body(i, idx):
        samp_ref[...] = jnp.where(iota_np == i, idx[:, :, 0], samp_ref[...])
        mask = (flat4 == idx[:, :, :, None]).astype(jnp.float32)
        p = jnp.sum(pts * mask, axis=(2, 3), keepdims=True)
        p2_ref[...] = jnp.where(iota_np3 == i, p[:, :, :, 0], p2_ref[...])
        diff = pts - p
        sq = diff * diff
        d2 = (sq[:, 0] + sq[:, 1]) + sq[:, 2]
        nd = jnp.minimum(d2, dist_ref[...])
        dist_ref[...] = nd
        m = jnp.max(nd, axis=(1, 2), keepdims=True)
        cand = jnp.where(nd == m, flat3, n)
        return jnp.min(cand, axis=(1, 2), keepdims=True).astype(i32)

    lax.fori_loop(0, n_points, body, jnp.zeros((b, 1, 1), i32))


def _fps(pts4):
    b, d, r, l = pts4.shape
    n_points = (r * l) // 4
    return pl.pallas_call(
        _fps_body,
        out_shape=(jax.ShapeDtypeStruct((b, n_points), jnp.int32),
                   jax.ShapeDtypeStruct((b, 3, n_points), jnp.float32)),
        scratch_shapes=[pltpu.VMEM((b, r, l), jnp.float32)],
    )(pts4)


def _knn_body(cent_ref, pts_ref, out_ref, work_ref):
    C = cent_ref.shape[1]
    n = pts_ref.shape[2]
    b_idx = pl.program_id(0)
    cx = cent_ref[0, :, 0:1]
    cy = cent_ref[0, :, 1:2]
    cz = cent_ref[0, :, 2:3]
    xr = pts_ref[0, 0:1, :]
    yr = pts_ref[0, 1:2, :]
    zr = pts_ref[0, 2:3, :]
    dx = cx - xr
    dy = cy - yr
    dz = cz - zr
    work_ref[...] = (dx * dx + dy * dy) + dz * dz
    lane = lax.broadcasted_iota(jnp.int32, (C, n), 1)
    cols = []
    for _ in range(_K):
        w = work_ref[...]
        m = jnp.min(w, axis=1, keepdims=True)
        idxv = jnp.min(jnp.where(w == m, lane, n), axis=1, keepdims=True)
        cols.append(idxv)
        work_ref[...] = jnp.where(lane == idxv, jnp.float32(3.0e38), w)
    out_ref[0] = jnp.concatenate(cols, axis=1) + b_idx * n


def _knn(p2c, pts_t, C=256):
    b, n_points, _ = p2c.shape
    n = pts_t.shape[2]
    return pl.pallas_call(
        _knn_body,
        grid=(b, n_points // C),
        in_specs=[
            pl.BlockSpec((1, C, 3), lambda bi, ci: (bi, ci, 0)),
            pl.BlockSpec((1, 3, n), lambda bi, ci: (bi, 0, 0)),
        ],
        out_specs=pl.BlockSpec((1, C, _K), lambda bi, ci: (bi, ci, 0)),
        out_shape=jax.ShapeDtypeStruct((b, n_points, _K), jnp.int32),
        scratch_shapes=[pltpu.VMEM((C, n), jnp.float32)],
    )(p2c, pts_t)


def _proj_body(f_ref, p_ref, wf_ref, wp_ref, g_ref):
    g = jnp.dot(f_ref[0], wf_ref[...], preferred_element_type=jnp.float32)
    g = g + jnp.dot(p_ref[0], wp_ref[...], preferred_element_type=jnp.float32)
    g_ref[0] = g


def _proj(feats, p_pad, wf, wp):
    b, n, F = feats.shape
    O = wf.shape[1]
    P = p_pad.shape[2]
    return pl.pallas_call(
        _proj_body,
        grid=(b,),
        in_specs=[
            pl.BlockSpec((1, n, F), lambda bi: (bi, 0, 0)),
            pl.BlockSpec((1, n, P), lambda bi: (bi, 0, 0)),
            pl.BlockSpec((F, O), lambda bi: (0, 0)),
            pl.BlockSpec((P, O), lambda bi: (0, 0)),
        ],
        out_specs=pl.BlockSpec((1, n, O), lambda bi: (bi, 0, 0)),
        out_shape=jax.ShapeDtypeStruct((b, n, O), jnp.float32),
    )(feats, p_pad, wf, wp)


def _sc_gather(table, idx):
    V, D = table.shape
    B = idx.shape[0]
    info = plsc.get_sparse_core_info()
    nw = info.num_cores * info.num_subcores
    CH = 128
    b_per_w = B // nw
    chunks = b_per_w // CH
    mesh = plsc.VectorSubcoreMesh(core_axis_name="c", subcore_axis_name="s")

    @functools.partial(
        pl.kernel, mesh=mesh,
        out_type=jax.ShapeDtypeStruct((B, D), jnp.float32),
        scratch_types=[
            pltpu.VMEM((CH,), jnp.int32),
            pltpu.VMEM((CH, D), jnp.float32),
            pltpu.SemaphoreType.DMA,
        ],
    )
    def gath(table_hbm, idx_hbm, out_hbm, idx_v, rows_v, sem):
        wid = lax.axis_index("s") * info.num_cores + lax.axis_index("c")
        base = wid * b_per_w

        def body(ci, carry):
            off = pl.multiple_of(base + ci * CH, CH)
            pltpu.sync_copy(idx_hbm.at[pl.ds(off, CH)], idx_v)
            pltpu.async_copy(table_hbm.at[idx_v], rows_v, sem).wait()
            pltpu.sync_copy(rows_v, out_hbm.at[pl.ds(off, CH)])
            return carry

        lax.fori_loop(0, chunks, body, 0)

    return gath(table, idx)


def _stats_body(x_ref, o_ref, s1_ref, s2_ref):
    i = pl.program_id(0)

    @pl.when(i == 0)
    def _():
        s1_ref[...] = jnp.zeros_like(s1_ref)
        s2_ref[...] = jnp.zeros_like(s2_ref)

    x = x_ref[...]
    s1_ref[...] += jnp.sum(x, axis=0, keepdims=True)
    s2_ref[...] += jnp.sum(x * x, axis=0, keepdims=True)

    @pl.when(i == pl.num_programs(0) - 1)
    def _():
        o_ref[0:1, :] = s1_ref[...]
        o_ref[1:2, :] = s2_ref[...]


def _stats(xg, R=4096):
    B, O = xg.shape
    return pl.pallas_call(
        _stats_body,
        grid=(B // R,),
        in_specs=[pl.BlockSpec((R, O), lambda i: (i, 0))],
        out_specs=pl.BlockSpec((2, O), lambda i: (0, 0)),
        out_shape=jax.ShapeDtypeStruct((2, O), jnp.float32),
        scratch_shapes=[pltpu.VMEM((1, O), jnp.float32),
                        pltpu.VMEM((1, O), jnp.float32)],
    )(xg)


def _bn_body(x_ref, st_ref, gam_ref, bet_ref, o_ref, *, n_tot):
    s1 = st_ref[0:1, :]
    s2 = st_ref[1:2, :]
    mean = s1 * (1.0 / n_tot)
    var = s2 * (1.0 / n_tot) - mean * mean
    inv = lax.rsqrt(var + 1e-5)
    a = inv * gam_ref[...]
    bconst = bet_ref[...] - mean * a
    y = x_ref[...] * a + bconst
    g = y * 0.5 * (1.0 + lax.erf(y * 0.7071067811865476))
    Cc = o_ref.shape[0]
    o_ref[...] = jnp.max(g.reshape(Cc, _K, g.shape[1]), axis=1)


def _bn(xg, stats, gam, bet, Cc=128):
    B, O = xg.shape
    nc = B // _K
    return pl.pallas_call(
        functools.partial(_bn_body, n_tot=float(B)),
        grid=(nc // Cc,),
        in_specs=[
            pl.BlockSpec((Cc * _K, O), lambda i: (i, 0)),
            pl.BlockSpec((2, O), lambda i: (0, 0)),
            pl.BlockSpec((1, O), lambda i: (0, 0)),
            pl.BlockSpec((1, O), lambda i: (0, 0)),
        ],
        out_specs=pl.BlockSpec((Cc, O), lambda i: (i, 0)),
        out_shape=jax.ShapeDtypeStruct((nc, O), jnp.float32),
    )(xg, stats, gam, bet)


def kernel(feats, points, W, b_fc, gamma, beta):
    b, n, F = feats.shape
    d = points.shape[2]
    O = W.shape[1]
    n_points = n // 4
    r = 8
    l = n // r

    pts_t = jnp.transpose(points, (0, 2, 1))
    sampled, p2t = _fps(pts_t.reshape(b, d, r, l))
    p2c = jnp.transpose(p2t, (0, 2, 1))
    knn = _knn(p2c, pts_t)

    wp = jnp.concatenate([W[:d], jnp.zeros((8 - d, O), W.dtype)], axis=0)
    p_pad = jnp.concatenate(
        [points, jnp.zeros((b, n, 8 - d), points.dtype)], axis=-1)
    G = _proj(feats, p_pad, W[d:], wp)

    xg = _sc_gather(G.reshape(b * n, O), knn.reshape(b * n_points * _K))
    st = _stats(xg)
    out = _bn(xg, st, gamma.reshape(1, O), beta.reshape(1, O))
    return out.reshape(b, n_points, O), p2c


# speedup vs baseline: 52.5816x; 1.0122x over previous
import functools

import jax
import jax.numpy as jnp
from jax import lax
from jax.experimental import pallas as pl
from jax.experimental.pallas import tpu as pltpu
from jax.experimental.pallas import tpu_sc as plsc

_K = 16


def _fps_body(pts_ref, cols_ref, samp_ref, p2_ref, dist_ref):
    b, d, r, l = pts_ref.shape
    n = r * l
    n_points = samp_ref.shape[1]
    i32 = jnp.int32
    flat3 = (lax.broadcasted_iota(i32, (1, r, l), 1) * l
             + lax.broadcasted_iota(i32, (1, r, l), 2))
    iota_np = lax.broadcasted_iota(i32, (b, n_points), 1)
    iota_np3 = lax.broadcasted_iota(i32, (1, 1, n_points), 2)
    dist_ref[...] = jnp.full((b, r, l), 1e10, jnp.float32)
    pts = pts_ref[...]

    def body(i, idx):
        samp_ref[...] = jnp.where(iota_np == i, idx[:, :, 0], samp_ref[...])
        rows = [cols_ref[bi, pl.ds(idx[bi, 0, 0], 1), 0:3] for bi in range(b)]
        p = jnp.concatenate(rows, axis=0)
        p2_ref[...] = jnp.where(iota_np3 == i, p[:, :, None], p2_ref[...])
        diff = pts - p[:, :, None, None]
        sq = diff * diff
        d2 = (sq[:, 0] + sq[:, 1]) + sq[:, 2]
        nd = jnp.minimum(d2, dist_ref[...])
        dist_ref[...] = nd
        m = jnp.max(nd, axis=(1, 2), keepdims=True)
        cand = jnp.where(nd == m, flat3, n)
        return jnp.min(cand, axis=(1, 2), keepdims=True).astype(i32)

    lax.fori_loop(0, n_points, body, jnp.zeros((b, 1, 1), i32))


def _fps(pts4, p_pad):
    b, d, r, l = pts4.shape
    n_points = (r * l) // 4
    return pl.pallas_call(
        _fps_body,
        out_shape=(jax.ShapeDtypeStruct((b, n_points), jnp.int32),
                   jax.ShapeDtypeStruct((b, 3, n_points), jnp.float32)),
        scratch_shapes=[pltpu.VMEM((b, r, l), jnp.float32)],
    )(pts4, p_pad)


def _knn_body(cent_ref, pts_ref, out_ref, work_ref):
    C = cent_ref.shape[1]
    n = pts_ref.shape[2]
    b_idx = pl.program_id(0)
    cx = cent_ref[0, :, 0:1]
    cy = cent_ref[0, :, 1:2]
    cz = cent_ref[0, :, 2:3]
    xr = pts_ref[0, 0:1, :]
    yr = pts_ref[0, 1:2, :]
    zr = pts_ref[0, 2:3, :]
    dx = cx - xr
    dy = cy - yr
    dz = cz - zr
    work_ref[...] = (dx * dx + dy * dy) + dz * dz
    lane = lax.broadcasted_iota(jnp.int32, (C, n), 1)
    cols = []
    for _ in range(_K):
        w = work_ref[...]
        m = jnp.min(w, axis=1, keepdims=True)
        idxv = jnp.min(jnp.where(w == m, lane, n), axis=1, keepdims=True)
        cols.append(idxv)
        work_ref[...] = jnp.where(lane == idxv, jnp.float32(3.0e38), w)
    out_ref[0] = jnp.concatenate(cols, axis=1) + b_idx * n


def _knn(p2c, pts_t, C=256):
    b, n_points, _ = p2c.shape
    n = pts_t.shape[2]
    return pl.pallas_call(
        _knn_body,
        grid=(b, n_points // C),
        in_specs=[
            pl.BlockSpec((1, C, 3), lambda bi, ci: (bi, ci, 0)),
            pl.BlockSpec((1, 3, n), lambda bi, ci: (bi, 0, 0)),
        ],
        out_specs=pl.BlockSpec((1, C, _K), lambda bi, ci: (bi, ci, 0)),
        out_shape=jax.ShapeDtypeStruct((b, n_points, _K), jnp.int32),
        scratch_shapes=[pltpu.VMEM((C, n), jnp.float32)],
    )(p2c, pts_t)


def _proj_body(f_ref, p_ref, wf_ref, wp_ref, g_ref):
    g = jnp.dot(f_ref[0], wf_ref[...], preferred_element_type=jnp.float32)
    g = g + jnp.dot(p_ref[0], wp_ref[...], preferred_element_type=jnp.float32)
    g_ref[0] = g


def _proj(feats, p_pad, wf, wp):
    b, n, F = feats.shape
    O = wf.shape[1]
    P = p_pad.shape[2]
    return pl.pallas_call(
        _proj_body,
        grid=(b,),
        in_specs=[
            pl.BlockSpec((1, n, F), lambda bi: (bi, 0, 0)),
            pl.BlockSpec((1, n, P), lambda bi: (bi, 0, 0)),
            pl.BlockSpec((F, O), lambda bi: (0, 0)),
            pl.BlockSpec((P, O), lambda bi: (0, 0)),
        ],
        out_specs=pl.BlockSpec((1, n, O), lambda bi: (bi, 0, 0)),
        out_shape=jax.ShapeDtypeStruct((b, n, O), jnp.float32),
    )(feats, p_pad, wf, wp)


def _sc_gather(table, idx):
    V, D = table.shape
    B = idx.shape[0]
    info = plsc.get_sparse_core_info()
    nw = info.num_cores * info.num_subcores
    CH = 128
    b_per_w = B // nw
    chunks = b_per_w // CH
    mesh = plsc.VectorSubcoreMesh(core_axis_name="c", subcore_axis_name="s")

    @functools.partial(
        pl.kernel, mesh=mesh,
        out_type=jax.ShapeDtypeStruct((B, D), jnp.float32),
        scratch_types=[
            pltpu.VMEM((CH,), jnp.int32),
            pltpu.VMEM((CH, D), jnp.float32),
            pltpu.SemaphoreType.DMA,
        ],
    )
    def gath(table_hbm, idx_hbm, out_hbm, idx_v, rows_v, sem):
        wid = lax.axis_index("s") * info.num_cores + lax.axis_index("c")
        base = wid * b_per_w

        def body(ci, carry):
            off = pl.multiple_of(base + ci * CH, CH)
            pltpu.sync_copy(idx_hbm.at[pl.ds(off, CH)], idx_v)
            pltpu.async_copy(table_hbm.at[idx_v], rows_v, sem).wait()
            pltpu.sync_copy(rows_v, out_hbm.at[pl.ds(off, CH)])
            return carry

        lax.fori_loop(0, chunks, body, 0)

    return gath(table, idx)


def _stats_body(x_ref, o_ref, s1_ref, s2_ref):
    i = pl.program_id(0)

    @pl.when(i == 0)
    def _():
        s1_ref[...] = jnp.zeros_like(s1_ref)
        s2_ref[...] = jnp.zeros_like(s2_ref)

    x = x_ref[...]
    s1_ref[...] += jnp.sum(x, axis=0, keepdims=True)
    s2_ref[...] += jnp.sum(x * x, axis=0, keepdims=True)

    @pl.when(i == pl.num_programs(0) - 1)
    def _():
        o_ref[0:1, :] = s1_ref[...]
        o_ref[1:2, :] = s2_ref[...]


def _stats(xg, R=4096):
    B, O = xg.shape
    return pl.pallas_call(
        _stats_body,
        grid=(B // R,),
        in_specs=[pl.BlockSpec((R, O), lambda i: (i, 0))],
        out_specs=pl.BlockSpec((2, O), lambda i: (0, 0)),
        out_shape=jax.ShapeDtypeStruct((2, O), jnp.float32),
        scratch_shapes=[pltpu.VMEM((1, O), jnp.float32),
                        pltpu.VMEM((1, O), jnp.float32)],
    )(xg)


def _bn_body(x_ref, st_ref, gam_ref, bet_ref, o_ref, *, n_tot):
    s1 = st_ref[0:1, :]
    s2 = st_ref[1:2, :]
    mean = s1 * (1.0 / n_tot)
    var = s2 * (1.0 / n_tot) - mean * mean
    inv = lax.rsqrt(var + 1e-5)
    a = inv * gam_ref[...]
    bconst = bet_ref[...] - mean * a
    y = x_ref[...] * a + bconst
    g = y * 0.5 * (1.0 + lax.erf(y * 0.7071067811865476))
    Cc = o_ref.shape[0]
    o_ref[...] = jnp.max(g.reshape(Cc, _K, g.shape[1]), axis=1)


def _bn(xg, stats, gam, bet, Cc=128):
    B, O = xg.shape
    nc = B // _K
    return pl.pallas_call(
        functools.partial(_bn_body, n_tot=float(B)),
        grid=(nc // Cc,),
        in_specs=[
            pl.BlockSpec((Cc * _K, O), lambda i: (i, 0)),
            pl.BlockSpec((2, O), lambda i: (0, 0)),
            pl.BlockSpec((1, O), lambda i: (0, 0)),
            pl.BlockSpec((1, O), lambda i: (0, 0)),
        ],
        out_specs=pl.BlockSpec((Cc, O), lambda i: (i, 0)),
        out_shape=jax.ShapeDtypeStruct((nc, O), jnp.float32),
    )(xg, stats, gam, bet)


def kernel(feats, points, W, b_fc, gamma, beta):
    b, n, F = feats.shape
    d = points.shape[2]
    O = W.shape[1]
    n_points = n // 4
    r = 8
    l = n // r

    pts_t = jnp.transpose(points, (0, 2, 1))
    p_pad = jnp.concatenate(
        [points, jnp.zeros((b, n, 8 - d), points.dtype)], axis=-1)
    sampled, p2t = _fps(pts_t.reshape(b, d, r, l), p_pad)
    p2c = jnp.transpose(p2t, (0, 2, 1))
    knn = _knn(p2c, pts_t)

    wp = jnp.concatenate([W[:d], jnp.zeros((8 - d, O), W.dtype)], axis=0)
    G = _proj(feats, p_pad, W[d:], wp)

    xg = _sc_gather(G.reshape(b * n, O), knn.reshape(b * n_points * _K))
    st = _stats(xg)
    out = _bn(xg, st, gamma.reshape(1, O), beta.reshape(1, O))
    return out.reshape(b, n_points, O), p2c
